# baseline (speedup 1.0000x reference)
import math
import functools

import jax
import jax.numpy as jnp
from jax.experimental import pallas as pl

EMB = 128
POOL_NUM = 3
KS = [0.8, 0.6, 0.5]
N_GRAPHS = 64


def _pool_sizes(n, ks):
    sizes = []
    cur = n
    for r in ks:
        cur = int(math.ceil(r * cur))
        sizes.append(cur)
    return sizes


def _gcn_conv(h, src, dst, keep, W, b):
    N = h.shape[0]
    ew = keep[src] * keep[dst]
    deg = jnp.zeros((N,), dtype=h.dtype).at[dst].add(ew) + keep
    deg = jnp.maximum(deg, 1e-6)
    dinv = 1.0 / jnp.sqrt(deg)
    coef = ew * dinv[src] * dinv[dst]
    msg = h[src] * coef[:, None]
    agg = jnp.zeros_like(h).at[dst].add(msg)
    agg = agg + h * (keep * dinv * dinv)[:, None]
    out = jax.nn.relu(agg @ W + b)
    return out * keep[:, None]


def _topk_pool(h, keep, p, k):
    score = (h @ p) / (jnp.linalg.norm(p) + 1e-12)
    masked = jnp.where(keep > 0.5, score, -1e30)
    _, idx = jax.lax.top_k(masked, k)
    new_keep = jnp.zeros_like(keep).at[idx].set(1.0)
    gate = jax.nn.sigmoid(score)
    h_new = h * gate[:, None] * new_keep[:, None]
    return h_new, new_keep


def _readout(h, keep, batch, num_graphs):
    s = jax.ops.segment_sum(h * keep[:, None], batch, num_segments=num_graphs)
    cnt = jax.ops.segment_sum(keep, batch, num_segments=num_graphs)
    mean = s / jnp.maximum(cnt, 1.0)[:, None]
    hm = jnp.where(keep[:, None] > 0.5, h, -1e30)
    mx = jax.ops.segment_max(hm, batch, num_segments=num_graphs)
    mx = jnp.where(mx < -1e29, 0.0, mx)
    return jnp.concatenate([mean, mx, s], axis=-1)


def _pred_kernel(gr_ref, w_ref, b_ref, o_ref):
    o_ref[...] = gr_ref[...] @ w_ref[...] + b_ref[...]


def kernel(x, edge_index, batch, W_enc, b_enc, W_in, b_in, W_d0, b_d0, W_d1,
           b_d1, W_d2, b_d2, W_u0, b_u0, W_u1, b_u1, W_u2, b_u2, p0, p1, p2,
           W_bot, b_bot, W_pred, b_pred):
    src = edge_index[0]
    dst = edge_index[1]
    seg = batch.astype(jnp.int32)
    N = x.shape[0]
    pool_sizes = _pool_sizes(N, KS)
    Wd = [W_d0, W_d1, W_d2]; bd = [b_d0, b_d1, b_d2]
    Wu = [W_u0, W_u1, W_u2]; bu = [b_u0, b_u1, b_u2]
    ps = [p0, p1, p2]
    keep0 = jnp.ones((N,), dtype=x.dtype)
    feat = x @ W_enc + b_enc
    ori_h = _gcn_conv(feat, src, dst, keep0, W_in, b_in)
    h = ori_h
    keeps = [keep0]
    down = []
    for i in range(POOL_NUM):
        h = _gcn_conv(h, src, dst, keeps[i], Wd[i], bd[i])
        down.append(h)
        h, knew = _topk_pool(h, keeps[i], ps[i], pool_sizes[i])
        keeps.append(knew)
    h = _gcn_conv(h, src, dst, keeps[POOL_NUM], W_bot, b_bot)
    hs = []
    for i in range(POOL_NUM):
        up = POOL_NUM - 1 - i
        h = _gcn_conv(h, src, dst, keeps[up], Wu[i], bu[i])
        h = h + down[up]
        hs.append((h, keeps[up]))
    h = h + ori_h
    hs.append((h, keep0))
    gr = jnp.concatenate([_readout(hh, kk, seg, N_GRAPHS) for hh, kk in hs],
                         axis=-1)
    out = pl.pallas_call(
        _pred_kernel,
        out_shape=jax.ShapeDtypeStruct((N_GRAPHS, W_pred.shape[1]),
                                       jnp.float32),
    )(gr, W_pred, b_pred)
    return out


# speedup vs baseline: 15.6874x; 15.6874x over previous
import functools
import math

import jax
import jax.numpy as jnp
from jax import lax
from jax.experimental import pallas as pl
from jax.experimental.pallas import tpu as pltpu
from jax.experimental.pallas import tpu_sc as plsc

N = 10000
E = 320000
D = 128
G = 64
POOL_NUM = 3
KS = [0.8, 0.6, 0.5]

NC = 2
NS = 16
NW = NC * NS
CHE = 128
NCHUNK = E // CHE
CPT = NCHUNK // NW
NEXTRA = NCHUNK - CPT * NW
NP = 10240
RPT = N // NS
SPT = NP // NS
CHR = 64

_f32 = jnp.float32
_i32 = jnp.int32


def _pool_sizes(n, ks):
    sizes = []
    cur = n
    for r in ks:
        cur = int(math.ceil(r * cur))
        sizes.append(cur)
    return sizes


POOL_SIZES = _pool_sizes(N, KS)

_MESH = plsc.VectorSubcoreMesh(core_axis_name="c", subcore_axis_name="s")


@functools.partial(
    pl.kernel,
    out_type=jax.ShapeDtypeStruct((NC, NP, D), _f32),
    mesh=_MESH,
    scratch_types=[
        pltpu.VMEM((CHE,), _i32),
        pltpu.VMEM((CHE,), _i32),
        pltpu.VMEM((CHE, D), _f32),
        pltpu.VMEM_SHARED((NP, D), _f32),
    ],
)
def _sc_edge_scatter(g_hbm, src_hbm, dst_hbm, out_hbm,
                     idxs, idxd, rows, acc):
    c = lax.axis_index("c")
    s = lax.axis_index("s")
    wid = c * NS + s

    zero = jnp.zeros((16,), _f32)

    def zrow(r, carry):
        for q in range(D // 16):
            rows[r, pl.ds(q * 16, 16)] = zero
        return carry

    lax.fori_loop(0, CHE, zrow, 0)
    base = s * SPT
    for t in range(SPT // CHE):
        pltpu.sync_copy(rows, acc.at[pl.ds(base + t * CHE, CHE)])
    plsc.subcore_barrier()

    def step_chunk(chunk_id):
        o = chunk_id * CHE
        pltpu.sync_copy(src_hbm.at[pl.ds(o, CHE)], idxs)
        pltpu.sync_copy(dst_hbm.at[pl.ds(o, CHE)], idxd)
        pltpu.sync_copy(g_hbm.at[idxs], rows)
        pltpu.sync_copy(rows, acc.at[idxd], add=True)

    def step(i, carry):
        step_chunk(wid * CPT + i)
        return carry

    lax.fori_loop(0, CPT, step, 0)

    @pl.when(wid < NEXTRA)
    def _():
        step_chunk(NW * CPT + wid)

    plsc.subcore_barrier()
    pltpu.sync_copy(acc.at[pl.ds(base, SPT)], out_hbm.at[c, pl.ds(base, SPT)])


@functools.partial(
    pl.kernel,
    out_type=jax.ShapeDtypeStruct((NC, NP), _f32),
    mesh=_MESH,
    scratch_types=[
        pltpu.VMEM((CHE,), _i32),
        pltpu.VMEM((CHE,), _i32),
        pltpu.VMEM((CHE,), _f32),
        pltpu.VMEM_SHARED((NP,), _f32),
    ],
)
def _sc_degree(keep_hbm, src_hbm, dst_hbm, out_hbm,
               idxs, idxd, vals, acc):
    c = lax.axis_index("c")
    s = lax.axis_index("s")
    wid = c * NS + s

    zero = jnp.zeros((16,), _f32)

    def zv(r, carry):
        vals[pl.ds(r * 16, 16)] = zero
        return carry

    lax.fori_loop(0, CHE // 16, zv, 0)
    for t in range(SPT // CHE):
        pltpu.sync_copy(vals.at[pl.ds(0, CHE)],
                        acc.at[pl.ds(s * SPT + t * CHE, CHE)])

    plsc.subcore_barrier()

    def step_chunk(chunk_id):
        o = chunk_id * CHE
        pltpu.sync_copy(src_hbm.at[pl.ds(o, CHE)], idxs)
        pltpu.sync_copy(dst_hbm.at[pl.ds(o, CHE)], idxd)
        pltpu.sync_copy(keep_hbm.at[idxs], vals)
        pltpu.sync_copy(vals, acc.at[idxd], add=True)

    def step(i, carry):
        step_chunk(wid * CPT + i)
        return carry

    lax.fori_loop(0, CPT, step, 0)

    @pl.when(wid < NEXTRA)
    def _():
        step_chunk(NW * CPT + wid)

    plsc.subcore_barrier()
    pltpu.sync_copy(acc.at[pl.ds(s * SPT, SPT)],
                    out_hbm.at[c, pl.ds(s * SPT, SPT)])


@functools.partial(
    pl.kernel,
    out_type=jax.ShapeDtypeStruct((4, G, D), _f32),
    mesh=_MESH,
    scratch_types=[
        pltpu.VMEM((128,), _i32),
        pltpu.VMEM((CHR, D), _f32),
        pltpu.VMEM((D,), _f32),
    ],
)
def _sc_segmax(hm_hbm, bounds_hbm, maxs_hbm, bvec, hbuf, macc):
    c = lax.axis_index("c")
    s = lax.axis_index("s")
    wid = c * NS + s

    pltpu.sync_copy(bounds_hbm, bvec)
    neg = jnp.full((16,), -1e30, _f32)

    for lvl in range(4):
        for g_loc in range(2):
            g = wid * 2 + g_loc
            st = bvec[pl.ds(g, 16)][0]
            en = bvec[pl.ds(g + 1, 16)][0]
            for q in range(D // 16):
                macc[pl.ds(q * 16, 16)] = neg
            a0 = lax.div(st, 8) * 8
            nch = lax.div(en - a0 + (CHR - 1), CHR)

            def chunk(i, carry):
                nominal = a0 + i * CHR
                c0 = jnp.minimum(nominal, N - CHR)
                lo_b = jnp.maximum(st, nominal)
                hi_b = jnp.minimum(en, nominal + CHR)
                pltpu.sync_copy(hm_hbm.at[lvl, pl.ds(c0, CHR)], hbuf)

                def rowf(j, carry2):
                    r = c0 + j

                    @pl.when((r >= lo_b) & (r < hi_b))
                    def _():
                        for q in range(D // 16):
                            sl = pl.ds(q * 16, 16)
                            macc[sl] = jnp.maximum(macc[sl], hbuf[j, sl])

                    return carry2

                return lax.fori_loop(0, CHR, rowf, carry)

            lax.fori_loop(0, nch, chunk, 0)
            pltpu.sync_copy(macc, maxs_hbm.at[lvl, g])


def _enc_body(x_ref, w_ref, b_ref, o_ref):
    o_ref[...] = jnp.dot(x_ref[...], w_ref[...],
                         preferred_element_type=_f32,
                         precision=lax.Precision.HIGHEST) + b_ref[...]


def _enc(x, w, b):
    return pl.pallas_call(
        _enc_body,
        out_shape=jax.ShapeDtypeStruct((N, D), _f32),
    )(x, w, b)


def _pre_body(h_ref, kd_ref, o_ref):
    o_ref[...] = kd_ref[...] * h_ref[...]


def _pre(h, kd):
    return pl.pallas_call(
        _pre_body,
        out_shape=jax.ShapeDtypeStruct((N, D), _f32),
    )(h, kd)


def _epi_body(s_ref, q_ref, kd_ref, keep_ref, w_ref, b_ref, add_ref,
              o_ref):
    agg = kd_ref[...] * (s_ref[0] + s_ref[1] + q_ref[...])
    out = jnp.dot(agg, w_ref[...], preferred_element_type=_f32,
                         precision=lax.Precision.HIGHEST) + b_ref[...]
    o_ref[...] = jnp.maximum(out, 0.0) * keep_ref[...] + add_ref[...]


_BR = 2000


def _epi(s, q, kd, keep, w, b, add):
    return pl.pallas_call(
        _epi_body,
        grid=(N // _BR,),
        in_specs=[
            pl.BlockSpec((NC, _BR, D), lambda r: (0, r, 0)),
            pl.BlockSpec((_BR, D), lambda r: (r, 0)),
            pl.BlockSpec((_BR, 1), lambda r: (r, 0)),
            pl.BlockSpec((_BR, 1), lambda r: (r, 0)),
            pl.BlockSpec((D, D), lambda r: (0, 0)),
            pl.BlockSpec((D,), lambda r: (0,)),
            pl.BlockSpec((_BR, D), lambda r: (r, 0)),
        ],
        out_specs=pl.BlockSpec((_BR, D), lambda r: (r, 0)),
        out_shape=jax.ShapeDtypeStruct((N, D), _f32),
    )(s, q, kd, keep, w, b, add)


def _kd_body(t_ref, keep_ref, o_ref):
    keep = keep_ref[...]
    deg = keep * (t_ref[0] + t_ref[1]) + keep
    deg = jnp.maximum(deg, 1e-6)
    o_ref[...] = keep * (1.0 / jnp.sqrt(deg))


def _kd(t, keep):
    return pl.pallas_call(
        _kd_body,
        out_shape=jax.ShapeDtypeStruct((N, 1), _f32),
    )(t, keep)


def _topk_body(h_ref, keep_ref, p_ref, hnew_ref, knew_ref, *, kk):
    p = p_ref[...]
    h = h_ref[...]
    score = jnp.dot(h, p.reshape(D, 1), preferred_element_type=_f32,
                         precision=lax.Precision.HIGHEST)
    score = score / (jnp.sqrt(jnp.sum(p * p)) + 1e-12)
    keep = keep_ref[...]
    masked = jnp.where(keep > 0.5, score, -1e30)
    bits = lax.bitcast_convert_type(masked, _i32)
    u = jnp.where(bits < 0, jnp.bitwise_xor(bits, _i32(0x7FFFFFFF)), bits)
    uu = lax.bitcast_convert_type(
        jnp.bitwise_xor(u, _i32(-2147483648)), jnp.uint32)

    def bitstep(i, T):
        b = jnp.uint32(1) << (jnp.uint32(31) - i.astype(jnp.uint32))
        cand = T | b
        cnt = jnp.sum((uu >= cand).astype(_i32))
        return jnp.where(cnt >= kk, cand, T)

    T = lax.fori_loop(0, 32, bitstep, jnp.uint32(0))
    m = jnp.sum((uu > T).astype(_i32))
    idx = lax.broadcasted_iota(_i32, (N, 1), 0)
    tie = uu == T

    def tstep(i, lohi):
        lo, hi = lohi
        mid = (lo + hi) // 2
        ccc = m + jnp.sum((tie & (idx <= mid)).astype(_i32))
        good = ccc >= kk
        return (jnp.where(good, lo, mid + 1), jnp.where(good, mid, hi))

    lo, _hi = lax.fori_loop(0, 14, tstep, (_i32(0), _i32(N - 1)))
    sel = (uu > T) | (tie & (idx <= lo))
    knew = sel.astype(_f32)
    gate = 1.0 / (1.0 + jnp.exp(-score))
    hnew_ref[...] = h * gate * knew
    knew_ref[...] = knew


def _topk(h, keep, p, kk):
    return pl.pallas_call(
        functools.partial(_topk_body, kk=kk),
        out_shape=[
            jax.ShapeDtypeStruct((N, D), _f32),
            jax.ShapeDtypeStruct((N, 1), _f32),
        ],
    )(h, keep, p)


def _bounds_body(b_ref, o_ref):
    batch = b_ref[...]
    cols = lax.broadcasted_iota(_i32, (1, 128), 1)
    o_ref[...] = jnp.sum((batch < cols).astype(_i32), axis=0, keepdims=True)


def _bounds(batch2d):
    return pl.pallas_call(
        _bounds_body,
        out_shape=jax.ShapeDtypeStruct((1, 128), _i32),
    )(batch2d)


def _rtc_body(batch_ref, h_ref, k_ref, sums_ref, cnts_ref, hm_ref):
    b = batch_ref[...]
    cols = lax.broadcasted_iota(_i32, (1, G), 1)
    oh = (b == cols).astype(_f32)
    h = h_ref[0]
    k = k_ref[0]
    dn = (((0,), (0,)), ((), ()))
    sums_ref[0] = lax.dot_general(oh, h * k, dn, preferred_element_type=_f32, precision=lax.Precision.HIGHEST)
    cnts_ref[0] = lax.dot_general(oh, k, dn, preferred_element_type=_f32, precision=lax.Precision.HIGHEST)
    hm_ref[0] = jnp.where(k > 0.5, h, -1e30)


def _rtc(batch2d, hstack, kstack):
    return pl.pallas_call(
        _rtc_body,
        grid=(4,),
        in_specs=[
            pl.BlockSpec((N, 1), lambda l: (0, 0)),
            pl.BlockSpec((1, N, D), lambda l: (l, 0, 0)),
            pl.BlockSpec((1, N, 1), lambda l: (l, 0, 0)),
        ],
        out_specs=[
            pl.BlockSpec((1, G, D), lambda l: (l, 0, 0)),
            pl.BlockSpec((1, G, 1), lambda l: (l, 0, 0)),
            pl.BlockSpec((1, N, D), lambda l: (l, 0, 0)),
        ],
        out_shape=[
            jax.ShapeDtypeStruct((4, G, D), _f32),
            jax.ShapeDtypeStruct((4, G, 1), _f32),
            jax.ShapeDtypeStruct((4, N, D), _f32),
        ],
    )(batch2d, hstack, kstack)


def _final_body(sums_ref, maxs_ref, cnts_ref, w_ref, b_ref, o_ref):
    nt = o_ref.shape[1]
    acc = jnp.zeros((G, nt), _f32) + b_ref[...]
    for lvl in range(4):
        sv = sums_ref[lvl]
        mx = maxs_ref[lvl]
        cnt = cnts_ref[lvl]
        mean = sv / jnp.maximum(cnt, 1.0)
        mxf = jnp.where(mx < -1e29, 0.0, mx)
        base = lvl * 3 * D
        acc = acc + jnp.dot(mean, w_ref[pl.ds(base, D), :],
                            preferred_element_type=_f32,
                         precision=lax.Precision.HIGHEST)
        acc = acc + jnp.dot(mxf, w_ref[pl.ds(base + D, D), :],
                            preferred_element_type=_f32,
                         precision=lax.Precision.HIGHEST)
        acc = acc + jnp.dot(sv, w_ref[pl.ds(base + 2 * D, D), :],
                            preferred_element_type=_f32,
                         precision=lax.Precision.HIGHEST)
    o_ref[...] = acc


def _add_body(a_ref, b_ref, o_ref):
    o_ref[...] = a_ref[...] + b_ref[...]


def _add(a, b):
    return pl.pallas_call(
        _add_body,
        out_shape=jax.ShapeDtypeStruct((N, D), _f32),
    )(a, b)


def _final(sums, maxs, cnts, w, b):
    return pl.pallas_call(
        _final_body,
        out_shape=jax.ShapeDtypeStruct((G, w.shape[1]), _f32),
    )(sums, maxs, cnts, w, b)


def _conv(h, src, dst, kd, keep2d, W, b, add):
    q = _pre(h, kd)
    s = _sc_edge_scatter(q, src, dst)
    return _epi(s[:, :N], q, kd, keep2d, W, b, add)


def kernel(x, edge_index, batch, W_enc, b_enc, W_in, b_in, W_d0, b_d0, W_d1,
           b_d1, W_d2, b_d2, W_u0, b_u0, W_u1, b_u1, W_u2, b_u2, p0, p1, p2,
           W_bot, b_bot, W_pred, b_pred):
    src = edge_index[0].astype(_i32)
    dst = edge_index[1].astype(_i32)
    batch2d = batch.astype(_i32).reshape(N, 1)
    zeros_nd = jnp.zeros((N, D), _f32)

    Wd = [W_d0, W_d1, W_d2]
    bd = [b_d0, b_d1, b_d2]
    Wu = [W_u0, W_u1, W_u2]
    bu = [b_u0, b_u1, b_u2]
    ps = [p0, p1, p2]

    def padN(v):
        return jnp.pad(v, (0, NP - N))

    keep0 = padN(jnp.ones((N,), _f32))
    keep0_2d = jnp.ones((N, 1), _f32)

    def level_kd(keep1d_p, keep2d):
        t = _sc_degree(keep1d_p, src, dst)
        return _kd(t[:, :N].reshape(NC, N, 1), keep2d)

    kd0 = level_kd(keep0, keep0_2d)

    feat = _enc(x, W_enc, b_enc)
    ori_h = _conv(feat, src, dst, kd0, keep0_2d, W_in, b_in, zeros_nd)

    h = ori_h
    keeps2d = [keep0_2d]
    keeps1d = [keep0]
    kds = [kd0]
    down = []
    for i in range(POOL_NUM):
        h = _conv(h, src, dst, kds[i], keeps2d[i], Wd[i], bd[i], zeros_nd)
        down.append(h)
        h, knew2d = _topk(h, keeps2d[i], ps[i], POOL_SIZES[i])
        knew1d = padN(knew2d.reshape(N))
        keeps2d.append(knew2d)
        keeps1d.append(knew1d)
        kds.append(level_kd(knew1d, knew2d))

    h = _conv(h, src, dst, kds[POOL_NUM], keeps2d[POOL_NUM], W_bot, b_bot,
              zeros_nd)

    hs = []
    for i in range(POOL_NUM):
        up = POOL_NUM - 1 - i
        h = _conv(h, src, dst, kds[up], keeps2d[up], Wu[i], bu[i], down[up])
        hs.append((h, keeps2d[up]))
    hlast = _add(h, ori_h)
    hs.append((hlast, keep0_2d))

    hstack = jnp.stack([hh for hh, _ in hs])
    kstack = jnp.stack([kk for _, kk in hs])
    sums, cnts, hm = _rtc(batch2d, hstack, kstack)
    bounds = _bounds(batch2d).reshape(128)
    maxs = _sc_segmax(hm, bounds)

    return _final(sums, maxs, cnts, W_pred, b_pred)
